# baseline (speedup 1.0000x reference)
import functools

import jax
import jax.numpy as jnp
import numpy as np
from jax import lax
from jax.experimental import pallas as pl
from jax.experimental.pallas import tpu as pltpu
from jax.experimental.pallas import tpu_sc as plsc

_BATCH = 4
_RADIUS = 240
_ANGLE = 180
_HEIGHT = 32
_RKEEP = 120
_NPTS = 480000
_CUT = 8

_NTILES = 32
_PTS_PER_TILE = _NPTS // _NTILES
_CHUNK0 = 7504
_CHUNK1 = _PTS_PER_TILE - _CHUNK0
_ROW_W = _ANGLE * _HEIGHT
_VL_WORDS = _BATCH * _RADIUS * _ROW_W
_KEEP_CHUNK = (_RKEEP * _ROW_W) // 8
_SWAP_ROWS = _RADIUS - _RKEEP
_AREA_THRESH = (22, 45, 67, 90, 112, 135, 157)


_PERMS = np.array([
    [1, 3, 0, 2], [2, 0, 3, 1], [0, 1, 2, 3], [3, 2, 0, 1],
    [1, 3, 2, 0], [3, 1, 2, 0], [1, 0, 3, 2], [0, 2, 1, 3],
], dtype=np.int32)


def _build_tables():
    inv = np.zeros((_CUT, _BATCH), np.int32)
    for a in range(_CUT):
        for b in range(_BATCH):
            inv[a, _PERMS[a, b]] = b
    return np.concatenate([inv.reshape(-1), _PERMS.reshape(-1)])


def _sc_call(coords_flat, vl_flat, tabs):
    mesh = plsc.VectorSubcoreMesh(
        core_axis_name="c", subcore_axis_name="s", num_cores=2, num_subcores=16)

    @functools.partial(
        pl.kernel,
        out_type=(
            jax.ShapeDtypeStruct((_NPTS * 3,), jnp.int32),
            jax.ShapeDtypeStruct((_VL_WORDS,), jnp.int32),
        ),
        mesh=mesh,
        compiler_params=pltpu.CompilerParams(needs_layout_passes=False),
        scratch_types=(
            pltpu.VMEM((64,), jnp.int32),
            pltpu.VMEM((_CHUNK0 * 3,), jnp.int32),
            pltpu.VMEM((_SWAP_ROWS * 23 * _HEIGHT,), jnp.int32),
            pltpu.SemaphoreType.DMA,
        ),
    )
    def body(coords_hbm, vl_hbm, tabs_hbm, out_coords, out_vl,
             tbuf, cbuf, vbuf, sem_s):
        wid = lax.axis_index("s") * 2 + lax.axis_index("c")
        pltpu.sync_copy(tabs_hbm, tbuf)

        def swap_copy(inbound):
            for t in range(_NTILES):
                if t < 16:
                    k, b = t // 4, t % 4
                    area, st, w = 2 * k, 45 * k, 22
                else:
                    k, b = (t - 16) // 4, (t - 16) % 4
                    area, st, w = 2 * k + 1, 45 * k + 22, 23
                src = int(_PERMS[area, b])
                rw = w * _HEIGHT
                in_base = (src * _RADIUS + _RKEEP) * _ROW_W + st * _HEIGHT
                out_base = (b * _RADIUS + _RKEEP) * _ROW_W + st * _HEIGHT

                @pl.when(wid == t)
                def _(in_base=in_base, out_base=out_base, rw=rw):
                    @pl.loop(0, _SWAP_ROWS)
                    def _(r):
                        if inbound:
                            pltpu.async_copy(
                                vl_hbm.at[pl.ds(in_base + r * _ROW_W, rw)],
                                vbuf.at[pl.ds(r * rw, rw)], sem_s)
                        else:
                            pltpu.async_copy(
                                vbuf.at[pl.ds(r * rw, rw)],
                                out_vl.at[pl.ds(out_base + r * _ROW_W, rw)],
                                sem_s)

        def swap_drain():
            def wait(w):
                n = _SWAP_ROWS * w * _HEIGHT
                pltpu.make_async_copy(vl_hbm.at[pl.ds(0, n)],
                                      vbuf.at[pl.ds(0, n)], sem_s).wait()

            @pl.when(wid < 16)
            def _():
                wait(22)

            @pl.when(wid >= 16)
            def _():
                wait(23)

        swap_copy(inbound=True)

        koff = (wid // 8) * (_RADIUS * _ROW_W) + (wid % 8) * _KEEP_CHUNK
        kq = _KEEP_CHUNK // 4

        @pl.loop(0, 4)
        def _(i):
            off = koff + i * kq
            pltpu.sync_copy(vl_hbm.at[pl.ds(off, kq)], cbuf.at[pl.ds(0, kq)])
            pltpu.sync_copy(cbuf.at[pl.ds(0, kq)], out_vl.at[pl.ds(off, kq)])

        swap_drain()
        swap_copy(inbound=False)

        cbase = wid * (_PTS_PER_TILE * 3)
        iota = lax.iota(jnp.int32, 16)

        def relabel(c0, c1, c2):
            area = (c2 >= _AREA_THRESH[0]).astype(jnp.int32)
            for t in _AREA_THRESH[1:]:
                area = area + (c2 >= t).astype(jnp.int32)
            lut = plsc.load_gather(tbuf, [area * 4 + c0])
            return jnp.where(c1 >= _RKEEP, lut, c0)

        def process(npts, off, nfull, rem):
            pltpu.sync_copy(coords_hbm.at[pl.ds(cbase + off, npts * 3)],
                            cbuf.at[pl.ds(0, npts * 3)])

            @pl.loop(0, nfull)
            def _(g):
                idx = g * 48 + iota * 3
                c0 = plsc.load_gather(cbuf, [idx])
                c1 = plsc.load_gather(cbuf, [idx + 1])
                c2 = plsc.load_gather(cbuf, [idx + 2])
                plsc.store_scatter(cbuf, [idx], relabel(c0, c1, c2))

            if rem:
                mask = iota < rem
                idx = jnp.where(mask, nfull * 48 + iota * 3, 0)
                c0 = plsc.load_gather(cbuf, [idx], mask=mask)
                c1 = plsc.load_gather(cbuf, [idx + 1], mask=mask)
                c2 = plsc.load_gather(cbuf, [idx + 2], mask=mask)
                plsc.store_scatter(cbuf, [idx], relabel(c0, c1, c2), mask=mask)

            pltpu.sync_copy(cbuf.at[pl.ds(0, npts * 3)],
                            out_coords.at[pl.ds(cbase + off, npts * 3)])

        process(_CHUNK0, 0, _CHUNK0 // 16, 0)
        process(_CHUNK1, _CHUNK0 * 3, _CHUNK1 // 16, _CHUNK1 % 16)

        swap_drain()

    return body(coords_flat, vl_flat, tabs)


@jax.jit
def _impl(point_feature_, point_coord_, voxel_label_):
    tabs = jnp.asarray(_build_tables(), dtype=jnp.int32)
    out_c, out_v = _sc_call(point_coord_.reshape(-1), voxel_label_.reshape(-1),
                            tabs)
    return (point_feature_,
            out_c.reshape(_NPTS, 3),
            out_v.reshape(_BATCH, _RADIUS, _ANGLE, _HEIGHT))


def kernel(point_feature_, point_coord_, voxel_label_):
    return _impl(point_feature_, point_coord_, voxel_label_)
